# baseline (speedup 1.0000x reference)
import jax
import jax.numpy as jnp
from jax import lax
from jax.experimental import pallas as pl
from jax.experimental.pallas import tpu as pltpu
from jax.experimental.pallas import tpu_sc as plsc

N = 10000
E = 320000
D = 128
DH = D // 2

NUM_CORES = 2
NUM_SUBCORES = 16
K = 80
NCHUNK = 252
E_PAD = NUM_SUBCORES * NCHUNK * K
NP = 10240
ROWS_PER_TILE = NP // NUM_SUBCORES

NEB = 6
NRB = 3
NMB = 2


def _sc_body(fT_hbm, edata_hbm, out_hbm, acc_sh, *bufs):
    ebuf = bufs[0:NEB]
    rows = bufs[NEB:NEB + NRB]
    msg = bufs[NEB + NRB:NEB + NRB + NMB]
    sems = bufs[NEB + NRB + NMB:]
    esem = sems[0:NEB]
    gsem = sems[NEB:NEB + NRB]
    ssem = sems[NEB + NRB:NEB + NRB + NMB]
    zsem = sems[NEB + NRB + NMB]

    c = lax.axis_index("c")
    s = lax.axis_index("s")
    ed_s = edata_hbm.at[s]
    fT_c = fT_hbm.at[c]
    last = NCHUNK - 1

    def load_edata(b, slot):
        pltpu.async_copy(ed_s.at[b], ebuf[slot], esem[slot])

    def wait_edata(slot):
        pltpu.make_async_copy(ed_s.at[0], ebuf[slot], esem[slot]).wait()

    def gather(b_slot, r):
        pltpu.async_copy(fT_c.at[ebuf[b_slot].at[0]], rows[r], gsem[r])

    def wait_gather(r):
        pltpu.make_async_copy(fT_c.at[ebuf[0].at[0]], rows[r], gsem[r]).wait()

    def scatter(slot_e, p):
        pltpu.async_copy(msg[p], acc_sh.at[ebuf[slot_e].at[1]], ssem[p],
                         add=True)

    def wait_scatter(p):
        pltpu.make_async_copy(msg[p], acc_sh.at[ebuf[0].at[1]], ssem[p]
                              ).wait()

    def compute(slot_e, r, p):
        rw = rows[r]
        mg = msg[p]

        def group_body(j, _):
            w16 = plsc.bitcast(ebuf[slot_e][2, pl.ds(j * 16, 16)],
                               jnp.float32)
            for kk in range(16):
                k = j * 16 + kk
                w = w16[kk]
                for dd in range(DH // 16):
                    v = rw[k, pl.ds(dd * 16, 16)]
                    m1 = v * w
                    mg[k, pl.ds(dd * 16, 16)] = m1
                    mg[k, pl.ds(DH + dd * 16, 16)] = m1 * v
            return _
        lax.fori_loop(0, K // 16, group_body, 0)

    for slot in range(4):
        load_edata(slot, slot)

    def zrow(r_, _):
        for dd in range(D // 16):
            msg[0][r_, pl.ds(dd * 16, 16)] = jnp.zeros((16,), jnp.float32)
        return _
    lax.fori_loop(0, K, zrow, 0)
    row0 = s * ROWS_PER_TILE
    zcps = [
        pltpu.async_copy(msg[0], acc_sh.at[pl.ds(row0 + j * K, K)], zsem)
        for j in range(ROWS_PER_TILE // K)
    ]
    for z in zcps:
        z.wait()
    plsc.subcore_barrier()

    for u in range(3):
        wait_edata(u)
        gather(u, u)

    def six_body(i6, _):
        b0 = i6 * 6
        for u in range(6):
            b = b0 + u
            r = u % 3
            p = u % 2
            wait_gather(r)
            if u >= 2:
                wait_scatter(p)
            else:
                @pl.when(i6 > 0)
                def _ws():
                    wait_scatter(p)
            load_edata(jnp.minimum(b + 4, last), (u + 4) % 6)
            compute(u, r, p)
            scatter(u, p)
            wait_edata((u + 3) % 6)
            gather((u + 3) % 6, r)
        return _

    lax.fori_loop(0, NCHUNK // 6, six_body, 0)

    for r in range(3):
        wait_gather(r)
    wait_scatter(0)
    wait_scatter(1)
    wait_edata(3)

    plsc.subcore_barrier()
    pltpu.sync_copy(acc_sh.at[pl.ds(row0, ROWS_PER_TILE)],
                    out_hbm.at[c].at[pl.ds(row0, ROWS_PER_TILE)])


@jax.jit
def _spmm_sc(fT, edata):
    mesh = plsc.VectorSubcoreMesh(core_axis_name="c", subcore_axis_name="s")
    run = pl.kernel(
        _sc_body,
        out_type=jax.ShapeDtypeStruct((NUM_CORES, NP, D), jnp.float32),
        mesh=mesh,
        scratch_types=(
            [pltpu.VMEM_SHARED((NP, D), jnp.float32)]
            + [pltpu.VMEM((3, K), jnp.int32)] * NEB
            + [pltpu.VMEM((K, DH), jnp.float32)] * NRB
            + [pltpu.VMEM((K, D), jnp.float32)] * NMB
            + [pltpu.SemaphoreType.DMA] * (NEB + NRB + NMB + 1)
        ),
        compiler_params=pltpu.CompilerParams(use_tc_tiling_on_sc=False,
                                             needs_layout_passes=False),
    )
    return run(fT, edata)


def _tc_body(f_ref, acc_ref, w1_ref, wsc_ref, b1_ref, b2_ref, o_ref):
    y = jnp.dot(f_ref[...], w1_ref[...], preferred_element_type=jnp.float32)
    y += jnp.dot(acc_ref[0], wsc_ref[0], preferred_element_type=jnp.float32)
    y += jnp.dot(acc_ref[1], wsc_ref[1], preferred_element_type=jnp.float32)
    o_ref[...] = y + b1_ref[...] + b2_ref[...]


@jax.jit
def _epilogue_tc(features, acc, W1, Wsc, b1, b2):
    R = 1000
    grid = (N // R,)
    return pl.pallas_call(
        _tc_body,
        grid=grid,
        in_specs=[
            pl.BlockSpec((R, D), lambda i: (i, 0)),
            pl.BlockSpec((NUM_CORES, R, D), lambda i: (0, i, 0)),
            pl.BlockSpec((D, D), lambda i: (0, 0)),
            pl.BlockSpec((NUM_CORES, D, D), lambda i: (0, 0, 0)),
            pl.BlockSpec((1, D), lambda i: (0, 0)),
            pl.BlockSpec((1, D), lambda i: (0, 0)),
        ],
        out_specs=pl.BlockSpec((R, D), lambda i: (i, 0)),
        out_shape=jax.ShapeDtypeStruct((N, D), jnp.float32),
    )(features, acc, W1, Wsc, b1, b2)


def kernel(features, edge_index, edge_weight, W1, b1, W2, b2):
    fT = features.reshape(N, NUM_CORES, DH).transpose(1, 0, 2)
    wbits = lax.bitcast_convert_type(edge_weight, jnp.int32)
    packed = jnp.stack([edge_index[0], edge_index[1], wbits], axis=0)
    packed = jnp.pad(packed, ((0, 0), (0, E_PAD - E)))
    edata = (packed.reshape(3, NUM_SUBCORES, NCHUNK, K)
             .transpose(1, 2, 0, 3))
    acc = _spmm_sc(fT, edata)
    Wsc = jnp.stack([
        jnp.concatenate([W1[:DH], W2[:DH]], axis=0),
        jnp.concatenate([W1[DH:], W2[DH:]], axis=0),
    ])
    return _epilogue_tc(features, acc, W1, Wsc,
                        b1.reshape(1, D), b2.reshape(1, D))


# speedup vs baseline: 1.3306x; 1.3306x over previous
import jax
import jax.numpy as jnp
from jax import lax
from jax.experimental import pallas as pl
from jax.experimental.pallas import tpu as pltpu
from jax.experimental.pallas import tpu_sc as plsc

N = 10000
E = 320000
D = 128
DH = D // 2

NUM_CORES = 2
NUM_SUBCORES = 16
K = 80
NCHUNK = 252
E_PAD = NUM_SUBCORES * NCHUNK * K
NP = 10240
ROWS_PER_TILE = NP // NUM_SUBCORES

NEB = 6
NRB = 3
NMB = 2


def _sc_body(fT_hbm, edata_hbm, out_hbm, acc_sh, *bufs):
    ebuf = bufs[0:NEB]
    rows = bufs[NEB:NEB + NRB]
    msg = bufs[NEB + NRB:NEB + NRB + NMB]
    sems = bufs[NEB + NRB + NMB:]
    esem = sems[0:NEB]
    gsem = sems[NEB:NEB + NRB]
    ssem = sems[NEB + NRB:NEB + NRB + NMB]
    zsem = sems[NEB + NRB + NMB]

    c = lax.axis_index("c")
    s = lax.axis_index("s")
    ed_s = edata_hbm.at[s]
    fT_c = fT_hbm.at[c]
    last = NCHUNK - 1

    def load_edata(b, slot):
        pltpu.async_copy(ed_s.at[b], ebuf[slot], esem[slot])

    def wait_edata(slot):
        pltpu.make_async_copy(ed_s.at[0], ebuf[slot], esem[slot]).wait()

    def gather(b_slot, r):
        pltpu.async_copy(fT_c.at[ebuf[b_slot].at[0]], rows[r], gsem[r])

    def wait_gather(r):
        pltpu.make_async_copy(fT_c.at[ebuf[0].at[0]], rows[r], gsem[r]).wait()

    def scatter(slot_e, p):
        pltpu.async_copy(msg[p], acc_sh.at[ebuf[slot_e].at[1]], ssem[p],
                         add=True)

    def wait_scatter(p):
        pltpu.make_async_copy(msg[p], acc_sh.at[ebuf[0].at[1]], ssem[p]
                              ).wait()

    def compute(slot_e, r, p):
        rw = rows[r]
        mg = msg[p]

        @plsc.parallel_loop(0, K // 16, unroll=2)
        def _groups(j):
            w16 = plsc.bitcast(ebuf[slot_e][2, pl.ds(j * 16, 16)],
                               jnp.float32)
            for kk in range(16):
                k = j * 16 + kk
                w = w16[kk]
                for dd in range(DH // 16):
                    v = rw[k, pl.ds(dd * 16, 16)]
                    m1 = v * w
                    mg[k, pl.ds(dd * 16, 16)] = m1
                    mg[k, pl.ds(DH + dd * 16, 16)] = m1 * v

    for slot in range(4):
        load_edata(slot, slot)

    def zrow(r_, _):
        for dd in range(D // 16):
            msg[0][r_, pl.ds(dd * 16, 16)] = jnp.zeros((16,), jnp.float32)
        return _
    lax.fori_loop(0, K, zrow, 0)
    row0 = s * ROWS_PER_TILE
    zcps = [
        pltpu.async_copy(msg[0], acc_sh.at[pl.ds(row0 + j * K, K)], zsem)
        for j in range(ROWS_PER_TILE // K)
    ]
    for z in zcps:
        z.wait()
    plsc.subcore_barrier()

    for u in range(3):
        wait_edata(u)
        gather(u, u)

    def six_body(i6, _):
        b0 = i6 * 6
        for u in range(6):
            b = b0 + u
            r = u % 3
            p = u % 2
            wait_gather(r)
            if u >= 2:
                wait_scatter(p)
            else:
                @pl.when(i6 > 0)
                def _ws():
                    wait_scatter(p)
            load_edata(jnp.minimum(b + 4, last), (u + 4) % 6)
            compute(u, r, p)
            scatter(u, p)
            wait_edata((u + 3) % 6)
            gather((u + 3) % 6, r)
        return _

    lax.fori_loop(0, NCHUNK // 6, six_body, 0)

    for r in range(3):
        wait_gather(r)
    wait_scatter(0)
    wait_scatter(1)
    wait_edata(3)

    plsc.subcore_barrier()
    pltpu.sync_copy(acc_sh.at[pl.ds(row0, ROWS_PER_TILE)],
                    out_hbm.at[c].at[pl.ds(row0, ROWS_PER_TILE)])


@jax.jit
def _spmm_sc(fT, edata):
    mesh = plsc.VectorSubcoreMesh(core_axis_name="c", subcore_axis_name="s")
    run = pl.kernel(
        _sc_body,
        out_type=jax.ShapeDtypeStruct((NUM_CORES, NP, D), jnp.float32),
        mesh=mesh,
        scratch_types=(
            [pltpu.VMEM_SHARED((NP, D), jnp.float32)]
            + [pltpu.VMEM((3, K), jnp.int32)] * NEB
            + [pltpu.VMEM((K, DH), jnp.float32)] * NRB
            + [pltpu.VMEM((K, D), jnp.float32)] * NMB
            + [pltpu.SemaphoreType.DMA] * (NEB + NRB + NMB + 1)
        ),
        compiler_params=pltpu.CompilerParams(use_tc_tiling_on_sc=False,
                                             needs_layout_passes=False),
    )
    return run(fT, edata)


def _tc_body(f_ref, acc_ref, w1_ref, wsc_ref, b1_ref, b2_ref, o_ref):
    y = jnp.dot(f_ref[...], w1_ref[...], preferred_element_type=jnp.float32)
    y += jnp.dot(acc_ref[0], wsc_ref[0], preferred_element_type=jnp.float32)
    y += jnp.dot(acc_ref[1], wsc_ref[1], preferred_element_type=jnp.float32)
    o_ref[...] = y + b1_ref[...] + b2_ref[...]


@jax.jit
def _epilogue_tc(features, acc, W1, Wsc, b1, b2):
    R = 1000
    grid = (N // R,)
    return pl.pallas_call(
        _tc_body,
        grid=grid,
        in_specs=[
            pl.BlockSpec((R, D), lambda i: (i, 0)),
            pl.BlockSpec((NUM_CORES, R, D), lambda i: (0, i, 0)),
            pl.BlockSpec((D, D), lambda i: (0, 0)),
            pl.BlockSpec((NUM_CORES, D, D), lambda i: (0, 0, 0)),
            pl.BlockSpec((1, D), lambda i: (0, 0)),
            pl.BlockSpec((1, D), lambda i: (0, 0)),
        ],
        out_specs=pl.BlockSpec((R, D), lambda i: (i, 0)),
        out_shape=jax.ShapeDtypeStruct((N, D), jnp.float32),
    )(features, acc, W1, Wsc, b1, b2)


def kernel(features, edge_index, edge_weight, W1, b1, W2, b2):
    fT = features.reshape(N, NUM_CORES, DH).transpose(1, 0, 2)
    wbits = lax.bitcast_convert_type(edge_weight, jnp.int32)
    packed = jnp.stack([edge_index[0], edge_index[1], wbits], axis=0)
    packed = jnp.pad(packed, ((0, 0), (0, E_PAD - E)))
    edata = (packed.reshape(3, NUM_SUBCORES, NCHUNK, K)
             .transpose(1, 2, 0, 3))
    acc = _spmm_sc(fT, edata)
    Wsc = jnp.stack([
        jnp.concatenate([W1[:DH], W2[:DH]], axis=0),
        jnp.concatenate([W1[DH:], W2[DH:]], axis=0),
    ])
    return _epilogue_tc(features, acc, W1, Wsc,
                        b1.reshape(1, D), b2.reshape(1, D))


# speedup vs baseline: 1.5197x; 1.1421x over previous
import jax
import jax.numpy as jnp
from jax import lax
from jax.experimental import pallas as pl
from jax.experimental.pallas import tpu as pltpu
from jax.experimental.pallas import tpu_sc as plsc

N = 10000
E = 320000
D = 128
DH = D // 2

NUM_CORES = 2
NUM_SUBCORES = 16
K = 80
NCHUNK = 252
E_PAD = NUM_SUBCORES * NCHUNK * K
NP = 10240
ROWS_PER_TILE = NP // NUM_SUBCORES

NEB = 6
NRB = 3
NMB = 2


def _sc_body(fT_hbm, edata_hbm, out_hbm, acc_sh, *bufs):
    ebuf = bufs[0:NEB]
    rows = bufs[NEB:NEB + NRB]
    msg = bufs[NEB + NRB:NEB + NRB + NMB]
    sems = bufs[NEB + NRB + NMB:]
    esem = sems[0:NEB]
    gsem = sems[NEB:NEB + NRB]
    ssem = sems[NEB + NRB:NEB + NRB + NMB]
    zsem = sems[NEB + NRB + NMB]

    c = lax.axis_index("c")
    s = lax.axis_index("s")
    ed_s = edata_hbm.at[s]
    fT_c = fT_hbm.at[c]
    last = NCHUNK - 1

    def load_edata(b, slot):
        pltpu.async_copy(ed_s.at[b], ebuf[slot], esem[slot])

    def wait_edata(slot):
        pltpu.make_async_copy(ed_s.at[0], ebuf[slot], esem[slot]).wait()

    def gather(b_slot, r):
        pltpu.async_copy(fT_c.at[ebuf[b_slot].at[0]], rows[r], gsem[r])

    def wait_gather(r):
        pltpu.make_async_copy(fT_c.at[ebuf[0].at[0]], rows[r], gsem[r]).wait()

    def scatter(slot_e, p):
        pltpu.async_copy(msg[p], acc_sh.at[ebuf[slot_e].at[1]], ssem[p],
                         add=True)

    def wait_scatter(p):
        pltpu.make_async_copy(msg[p], acc_sh.at[ebuf[0].at[1]], ssem[p]
                              ).wait()

    def compute(slot_e, r, p):
        rw = rows[r]
        mg = msg[p]

        @plsc.parallel_loop(0, K // 16, unroll=5)
        def _groups(j):
            w16 = plsc.bitcast(ebuf[slot_e][2, pl.ds(j * 16, 16)],
                               jnp.float32)
            for kk in range(16):
                k = j * 16 + kk
                w = w16[kk]
                for dd in range(DH // 16):
                    v = rw[k, pl.ds(dd * 16, 16)]
                    m1 = v * w
                    mg[k, pl.ds(dd * 16, 16)] = m1
                    mg[k, pl.ds(DH + dd * 16, 16)] = m1 * v

    for slot in range(4):
        load_edata(slot, slot)

    def zrow(r_, _):
        for dd in range(D // 16):
            msg[0][r_, pl.ds(dd * 16, 16)] = jnp.zeros((16,), jnp.float32)
        return _
    lax.fori_loop(0, K, zrow, 0)
    row0 = s * ROWS_PER_TILE
    zcps = [
        pltpu.async_copy(msg[0], acc_sh.at[pl.ds(row0 + j * K, K)], zsem)
        for j in range(ROWS_PER_TILE // K)
    ]
    for z in zcps:
        z.wait()
    plsc.subcore_barrier()

    for u in range(3):
        wait_edata(u)
        gather(u, u)

    def six_body(i6, _):
        b0 = i6 * 6
        for u in range(6):
            b = b0 + u
            r = u % 3
            p = u % 2
            wait_gather(r)
            if u >= 2:
                wait_scatter(p)
            else:
                @pl.when(i6 > 0)
                def _ws():
                    wait_scatter(p)
            load_edata(jnp.minimum(b + 4, last), (u + 4) % 6)
            compute(u, r, p)
            scatter(u, p)
            wait_edata((u + 3) % 6)
            gather((u + 3) % 6, r)
        return _

    lax.fori_loop(0, NCHUNK // 6, six_body, 0)

    for r in range(3):
        wait_gather(r)
    wait_scatter(0)
    wait_scatter(1)
    wait_edata(3)

    plsc.subcore_barrier()
    pltpu.sync_copy(acc_sh.at[pl.ds(row0, ROWS_PER_TILE)],
                    out_hbm.at[c].at[pl.ds(row0, ROWS_PER_TILE)])


@jax.jit
def _spmm_sc(fT, edata):
    mesh = plsc.VectorSubcoreMesh(core_axis_name="c", subcore_axis_name="s")
    run = pl.kernel(
        _sc_body,
        out_type=jax.ShapeDtypeStruct((NUM_CORES, NP, D), jnp.float32),
        mesh=mesh,
        scratch_types=(
            [pltpu.VMEM_SHARED((NP, D), jnp.float32)]
            + [pltpu.VMEM((3, K), jnp.int32)] * NEB
            + [pltpu.VMEM((K, DH), jnp.float32)] * NRB
            + [pltpu.VMEM((K, D), jnp.float32)] * NMB
            + [pltpu.SemaphoreType.DMA] * (NEB + NRB + NMB + 1)
        ),
        compiler_params=pltpu.CompilerParams(use_tc_tiling_on_sc=False,
                                             needs_layout_passes=False),
    )
    return run(fT, edata)


def _tc_body(f_ref, acc_ref, w1_ref, wsc_ref, b1_ref, b2_ref, o_ref):
    y = jnp.dot(f_ref[...], w1_ref[...], preferred_element_type=jnp.float32)
    y += jnp.dot(acc_ref[0], wsc_ref[0], preferred_element_type=jnp.float32)
    y += jnp.dot(acc_ref[1], wsc_ref[1], preferred_element_type=jnp.float32)
    o_ref[...] = y + b1_ref[...] + b2_ref[...]


@jax.jit
def _epilogue_tc(features, acc, W1, Wsc, b1, b2):
    R = 1000
    grid = (N // R,)
    return pl.pallas_call(
        _tc_body,
        grid=grid,
        in_specs=[
            pl.BlockSpec((R, D), lambda i: (i, 0)),
            pl.BlockSpec((NUM_CORES, R, D), lambda i: (0, i, 0)),
            pl.BlockSpec((D, D), lambda i: (0, 0)),
            pl.BlockSpec((NUM_CORES, D, D), lambda i: (0, 0, 0)),
            pl.BlockSpec((1, D), lambda i: (0, 0)),
            pl.BlockSpec((1, D), lambda i: (0, 0)),
        ],
        out_specs=pl.BlockSpec((R, D), lambda i: (i, 0)),
        out_shape=jax.ShapeDtypeStruct((N, D), jnp.float32),
    )(features, acc, W1, Wsc, b1, b2)


def kernel(features, edge_index, edge_weight, W1, b1, W2, b2):
    fT = features.reshape(N, NUM_CORES, DH).transpose(1, 0, 2)
    wbits = lax.bitcast_convert_type(edge_weight, jnp.int32)
    packed = jnp.stack([edge_index[0], edge_index[1], wbits], axis=0)
    packed = jnp.pad(packed, ((0, 0), (0, E_PAD - E)))
    edata = (packed.reshape(3, NUM_SUBCORES, NCHUNK, K)
             .transpose(1, 2, 0, 3))
    acc = _spmm_sc(fT, edata)
    Wsc = jnp.stack([
        jnp.concatenate([W1[:DH], W2[:DH]], axis=0),
        jnp.concatenate([W1[DH:], W2[DH:]], axis=0),
    ])
    return _epilogue_tc(features, acc, W1, Wsc,
                        b1.reshape(1, D), b2.reshape(1, D))


# speedup vs baseline: 1.5637x; 1.0289x over previous
import jax
import jax.numpy as jnp
from jax import lax
from jax.experimental import pallas as pl
from jax.experimental.pallas import tpu as pltpu
from jax.experimental.pallas import tpu_sc as plsc

N = 10000
E = 320000
D = 128
DH = D // 2

NUM_CORES = 2
NUM_SUBCORES = 16
K = 80
NCHUNK = 252
E_PAD = NUM_SUBCORES * NCHUNK * K
NP = 10240
ROWS_PER_TILE = NP // NUM_SUBCORES

NEB = 6
NRB = 3
NMB = 2


def _sc_body(fT_hbm, edata_hbm, out_hbm, acc_sh, *bufs):
    ebuf = bufs[0:NEB]
    rows = bufs[NEB:NEB + NRB]
    msg = bufs[NEB + NRB:NEB + NRB + NMB]
    sems = bufs[NEB + NRB + NMB:]
    esem = sems[0:NEB]
    gsem = sems[NEB:NEB + NRB]
    ssem = sems[NEB + NRB:NEB + NRB + NMB]
    zsem = sems[NEB + NRB + NMB]

    c = lax.axis_index("c")
    s = lax.axis_index("s")
    ed_s = edata_hbm.at[s]
    fT_c = fT_hbm.at[c]
    last = NCHUNK - 1

    def load_edata(b, slot):
        pltpu.async_copy(ed_s.at[b], ebuf[slot], esem[slot])

    def wait_edata(slot):
        pltpu.make_async_copy(ed_s.at[0], ebuf[slot], esem[slot]).wait()

    def gather(b_slot, r):
        pltpu.async_copy(fT_c.at[ebuf[b_slot].at[0]], rows[r], gsem[r])

    def wait_gather(r):
        pltpu.make_async_copy(fT_c.at[ebuf[0].at[0]], rows[r], gsem[r]).wait()

    def scatter(slot_e, p):
        pltpu.async_copy(msg[p], acc_sh.at[ebuf[slot_e].at[1]], ssem[p],
                         add=True)

    def wait_scatter(p):
        pltpu.make_async_copy(msg[p], acc_sh.at[ebuf[0].at[1]], ssem[p]
                              ).wait()

    def compute(slot_e, r, p):
        rw = rows[r]
        mg = msg[p]

        @plsc.parallel_loop(0, K // 16, unroll=5)
        def _groups(j):
            w16 = plsc.bitcast(ebuf[slot_e][2, pl.ds(j * 16, 16)],
                               jnp.float32)
            for kk in range(16):
                k = j * 16 + kk
                w = w16[kk]
                for blk in range(DH // 32):
                    vv = rw[k, pl.ds(blk * 32, 32)]
                    a, b = plsc.unpack(
                        vv, format=plsc.PackFormat.INTERLEAVED,
                        preferred_element_type=jnp.float32)
                    m1a = a * w
                    m1b = b * w
                    mg[k, pl.ds(blk * 32, 16)] = m1a
                    mg[k, pl.ds(blk * 32 + 16, 16)] = m1b
                    mg[k, pl.ds(DH + blk * 32, 16)] = m1a * a
                    mg[k, pl.ds(DH + blk * 32 + 16, 16)] = m1b * b

    for slot in range(4):
        load_edata(slot, slot)

    def zrow(r_, _):
        for dd in range(D // 16):
            msg[0][r_, pl.ds(dd * 16, 16)] = jnp.zeros((16,), jnp.float32)
        return _
    lax.fori_loop(0, K, zrow, 0)
    row0 = s * ROWS_PER_TILE
    zcps = [
        pltpu.async_copy(msg[0], acc_sh.at[pl.ds(row0 + j * K, K)], zsem)
        for j in range(ROWS_PER_TILE // K)
    ]
    for z in zcps:
        z.wait()
    plsc.subcore_barrier()

    for u in range(3):
        wait_edata(u)
        gather(u, u)

    def six_body(i6, _):
        b0 = i6 * 6
        for u in range(6):
            b = b0 + u
            r = u % 3
            p = u % 2
            wait_gather(r)
            if u >= 2:
                wait_scatter(p)
            else:
                @pl.when(i6 > 0)
                def _ws():
                    wait_scatter(p)
            load_edata(jnp.minimum(b + 4, last), (u + 4) % 6)
            compute(u, r, p)
            scatter(u, p)
            wait_edata((u + 3) % 6)
            gather((u + 3) % 6, r)
        return _

    lax.fori_loop(0, NCHUNK // 6, six_body, 0)

    for r in range(3):
        wait_gather(r)
    wait_scatter(0)
    wait_scatter(1)
    wait_edata(3)

    plsc.subcore_barrier()
    pltpu.sync_copy(acc_sh.at[pl.ds(row0, ROWS_PER_TILE)],
                    out_hbm.at[c].at[pl.ds(row0, ROWS_PER_TILE)])


@jax.jit
def _spmm_sc(fT, edata):
    mesh = plsc.VectorSubcoreMesh(core_axis_name="c", subcore_axis_name="s")
    run = pl.kernel(
        _sc_body,
        out_type=jax.ShapeDtypeStruct((NUM_CORES, NP, D), jnp.float32),
        mesh=mesh,
        scratch_types=(
            [pltpu.VMEM_SHARED((NP, D), jnp.float32)]
            + [pltpu.VMEM((3, K), jnp.int32)] * NEB
            + [pltpu.VMEM((K, DH), jnp.bfloat16)] * NRB
            + [pltpu.VMEM((K, D), jnp.float32)] * NMB
            + [pltpu.SemaphoreType.DMA] * (NEB + NRB + NMB + 1)
        ),
        compiler_params=pltpu.CompilerParams(use_tc_tiling_on_sc=False,
                                             needs_layout_passes=False),
    )
    return run(fT, edata)


def _tc_body(f_ref, acc_ref, w1_ref, wsc_ref, b1_ref, b2_ref, o_ref):
    y = jnp.dot(f_ref[...], w1_ref[...], preferred_element_type=jnp.float32)
    y += jnp.dot(acc_ref[0], wsc_ref[0], preferred_element_type=jnp.float32)
    y += jnp.dot(acc_ref[1], wsc_ref[1], preferred_element_type=jnp.float32)
    o_ref[...] = y + b1_ref[...] + b2_ref[...]


@jax.jit
def _epilogue_tc(features, acc, W1, Wsc, b1, b2):
    R = 1000
    grid = (N // R,)
    return pl.pallas_call(
        _tc_body,
        grid=grid,
        in_specs=[
            pl.BlockSpec((R, D), lambda i: (i, 0)),
            pl.BlockSpec((NUM_CORES, R, D), lambda i: (0, i, 0)),
            pl.BlockSpec((D, D), lambda i: (0, 0)),
            pl.BlockSpec((NUM_CORES, D, D), lambda i: (0, 0, 0)),
            pl.BlockSpec((1, D), lambda i: (0, 0)),
            pl.BlockSpec((1, D), lambda i: (0, 0)),
        ],
        out_specs=pl.BlockSpec((R, D), lambda i: (i, 0)),
        out_shape=jax.ShapeDtypeStruct((N, D), jnp.float32),
    )(features, acc, W1, Wsc, b1, b2)


def kernel(features, edge_index, edge_weight, W1, b1, W2, b2):
    fT = (features.astype(jnp.bfloat16)
          .reshape(N, NUM_CORES, DH).transpose(1, 0, 2))
    wbits = lax.bitcast_convert_type(edge_weight, jnp.int32)
    packed = jnp.stack([edge_index[0], edge_index[1], wbits], axis=0)
    packed = jnp.pad(packed, ((0, 0), (0, E_PAD - E)))
    edata = (packed.reshape(3, NUM_SUBCORES, NCHUNK, K)
             .transpose(1, 2, 0, 3))
    acc = _spmm_sc(fT, edata)
    perm = jnp.concatenate([
        jnp.arange(0, 32, 2), jnp.arange(1, 32, 2),
        jnp.arange(32, 64, 2), jnp.arange(33, 64, 2),
    ])
    Wsc = jnp.stack([
        jnp.concatenate([W1[:DH][perm], W2[:DH][perm]], axis=0),
        jnp.concatenate([W1[DH:][perm], W2[DH:][perm]], axis=0),
    ])
    return _epilogue_tc(features, acc, W1, Wsc,
                        b1.reshape(1, D), b2.reshape(1, D))


# speedup vs baseline: 1.5647x; 1.0007x over previous
import jax
import jax.numpy as jnp
from jax import lax
from jax.experimental import pallas as pl
from jax.experimental.pallas import tpu as pltpu
from jax.experimental.pallas import tpu_sc as plsc

N = 10000
E = 320000
D = 128
DH = D // 2

NUM_CORES = 2
NUM_SUBCORES = 16
K = 80
NCHUNK = 252
E_PAD = NUM_SUBCORES * NCHUNK * K
NP = 10240
ROWS_PER_TILE = NP // NUM_SUBCORES

NEB = 6
NRB = 2
NMB = 2


def _sc_body(fT_hbm, edata_hbm, out_hbm, acc_sh, fstage_sh, *bufs):
    ebuf = bufs[0:NEB]
    rows = bufs[NEB:NEB + NRB]
    msg = bufs[NEB + NRB:NEB + NRB + NMB]
    sems = bufs[NEB + NRB + NMB:]
    esem = sems[0:NEB]
    gsem = sems[NEB:NEB + NRB]
    ssem = sems[NEB + NRB:NEB + NRB + NMB]
    zsem = sems[NEB + NRB + NMB]

    c = lax.axis_index("c")
    s = lax.axis_index("s")
    ed_s = edata_hbm.at[s]
    fT_c = fT_hbm.at[c]
    last = NCHUNK - 1

    def load_edata(b, slot):
        pltpu.async_copy(ed_s.at[b], ebuf[slot], esem[slot])

    def wait_edata(slot):
        pltpu.make_async_copy(ed_s.at[0], ebuf[slot], esem[slot]).wait()

    def gather(b_slot, r):
        pltpu.async_copy(fstage_sh.at[ebuf[b_slot].at[0]], rows[r], gsem[r])

    def wait_gather(r):
        pltpu.make_async_copy(fstage_sh.at[ebuf[0].at[0]], rows[r], gsem[r]
                              ).wait()

    def scatter(slot_e, p):
        pltpu.async_copy(msg[p], acc_sh.at[ebuf[slot_e].at[1]], ssem[p],
                         add=True)

    def wait_scatter(p):
        pltpu.make_async_copy(msg[p], acc_sh.at[ebuf[0].at[1]], ssem[p]
                              ).wait()

    def compute(slot_e, r, p):
        rw = rows[r]
        mg = msg[p]

        @plsc.parallel_loop(0, K // 16, unroll=5)
        def _groups(j):
            w16 = plsc.bitcast(ebuf[slot_e][2, pl.ds(j * 16, 16)],
                               jnp.float32)
            for kk in range(16):
                k = j * 16 + kk
                w = w16[kk]
                for blk in range(DH // 32):
                    vv = rw[k, pl.ds(blk * 32, 32)]
                    a, b = plsc.unpack(
                        vv, format=plsc.PackFormat.INTERLEAVED,
                        preferred_element_type=jnp.float32)
                    m1a = a * w
                    m1b = b * w
                    mg[k, pl.ds(blk * 32, 16)] = m1a
                    mg[k, pl.ds(blk * 32 + 16, 16)] = m1b
                    mg[k, pl.ds(DH + blk * 32, 16)] = m1a * a
                    mg[k, pl.ds(DH + blk * 32 + 16, 16)] = m1b * b

    for slot in range(4):
        load_edata(slot, slot)

    row0 = s * ROWS_PER_TILE
    fcp = pltpu.async_copy(fT_c.at[pl.ds(row0, ROWS_PER_TILE)],
                           fstage_sh.at[pl.ds(row0, ROWS_PER_TILE)], zsem)

    def zrow(r_, _):
        for dd in range(D // 16):
            msg[0][r_, pl.ds(dd * 16, 16)] = jnp.zeros((16,), jnp.float32)
        return _
    lax.fori_loop(0, K, zrow, 0)
    fcp.wait()
    zcps = [
        pltpu.async_copy(msg[0], acc_sh.at[pl.ds(row0 + j * K, K)], zsem)
        for j in range(ROWS_PER_TILE // K)
    ]
    for z in zcps:
        z.wait()
    plsc.subcore_barrier()

    for u in range(2):
        wait_edata(u)
        gather(u, u)

    def six_body(i6, _):
        b0 = i6 * 6
        for u in range(6):
            b = b0 + u
            r = u % 2
            p = u % 2
            wait_gather(r)
            if u >= 2:
                wait_scatter(p)
            else:
                @pl.when(i6 > 0)
                def _ws():
                    wait_scatter(p)
            load_edata(jnp.minimum(b + 4, last), (u + 4) % 6)
            compute(u, r, p)
            scatter(u, p)
            wait_edata((u + 2) % 6)
            gather((u + 2) % 6, r)
        return _

    lax.fori_loop(0, NCHUNK // 6, six_body, 0)

    for r in range(2):
        wait_gather(r)
    wait_scatter(0)
    wait_scatter(1)
    wait_edata(2)
    wait_edata(3)

    plsc.subcore_barrier()
    pltpu.sync_copy(acc_sh.at[pl.ds(row0, ROWS_PER_TILE)],
                    out_hbm.at[c].at[pl.ds(row0, ROWS_PER_TILE)])


@jax.jit
def _spmm_sc(fT, edata):
    mesh = plsc.VectorSubcoreMesh(core_axis_name="c", subcore_axis_name="s")
    run = pl.kernel(
        _sc_body,
        out_type=jax.ShapeDtypeStruct((NUM_CORES, NP, D), jnp.float32),
        mesh=mesh,
        scratch_types=(
            [pltpu.VMEM_SHARED((NP, D), jnp.float32)]
            + [pltpu.VMEM_SHARED((NP, DH), jnp.bfloat16)]
            + [pltpu.VMEM((3, K), jnp.int32)] * NEB
            + [pltpu.VMEM((K, DH), jnp.bfloat16)] * NRB
            + [pltpu.VMEM((K, D), jnp.float32)] * NMB
            + [pltpu.SemaphoreType.DMA] * (NEB + NRB + NMB + 1)
        ),
        compiler_params=pltpu.CompilerParams(use_tc_tiling_on_sc=False,
                                             needs_layout_passes=False),
    )
    return run(fT, edata)


def _tc_body(f_ref, acc_ref, w1_ref, wsc_ref, b1_ref, b2_ref, o_ref):
    y = jnp.dot(f_ref[...], w1_ref[...], preferred_element_type=jnp.float32)
    y += jnp.dot(acc_ref[0], wsc_ref[0], preferred_element_type=jnp.float32)
    y += jnp.dot(acc_ref[1], wsc_ref[1], preferred_element_type=jnp.float32)
    o_ref[...] = y + b1_ref[...] + b2_ref[...]


@jax.jit
def _epilogue_tc(features, acc, W1, Wsc, b1, b2):
    R = 1000
    grid = (N // R,)
    return pl.pallas_call(
        _tc_body,
        grid=grid,
        in_specs=[
            pl.BlockSpec((R, D), lambda i: (i, 0)),
            pl.BlockSpec((NUM_CORES, R, D), lambda i: (0, i, 0)),
            pl.BlockSpec((D, D), lambda i: (0, 0)),
            pl.BlockSpec((NUM_CORES, D, D), lambda i: (0, 0, 0)),
            pl.BlockSpec((1, D), lambda i: (0, 0)),
            pl.BlockSpec((1, D), lambda i: (0, 0)),
        ],
        out_specs=pl.BlockSpec((R, D), lambda i: (i, 0)),
        out_shape=jax.ShapeDtypeStruct((N, D), jnp.float32),
    )(features, acc, W1, Wsc, b1, b2)


def kernel(features, edge_index, edge_weight, W1, b1, W2, b2):
    fT = (features.astype(jnp.bfloat16)
          .reshape(N, NUM_CORES, DH).transpose(1, 0, 2))
    fT = jnp.pad(fT, ((0, 0), (0, NP - N), (0, 0)))
    wbits = lax.bitcast_convert_type(edge_weight, jnp.int32)
    packed = jnp.stack([edge_index[0], edge_index[1], wbits], axis=0)
    packed = jnp.pad(packed, ((0, 0), (0, E_PAD - E)))
    edata = (packed.reshape(3, NUM_SUBCORES, NCHUNK, K)
             .transpose(1, 2, 0, 3))
    acc = _spmm_sc(fT, edata)
    perm = jnp.concatenate([
        jnp.arange(0, 32, 2), jnp.arange(1, 32, 2),
        jnp.arange(32, 64, 2), jnp.arange(33, 64, 2),
    ])
    Wsc = jnp.stack([
        jnp.concatenate([W1[:DH][perm], W2[:DH][perm]], axis=0),
        jnp.concatenate([W1[DH:][perm], W2[DH:][perm]], axis=0),
    ])
    return _epilogue_tc(features, acc, W1, Wsc,
                        b1.reshape(1, D), b2.reshape(1, D))


# speedup vs baseline: 1.9094x; 1.2203x over previous
import jax
import jax.numpy as jnp
from jax import lax
from jax.experimental import pallas as pl
from jax.experimental.pallas import tpu as pltpu
from jax.experimental.pallas import tpu_sc as plsc

N = 10000
E = 320000
D = 128
DH = D // 2

NUM_CORES = 2
NUM_SUBCORES = 16
K = 80
NCHUNK = 252
E_PAD = NUM_SUBCORES * NCHUNK * K
NP = 10240
ROWS_PER_TILE = NP // NUM_SUBCORES

NEB = 6
NRB = 2
NMB = 2


def _sc_body(fT_hbm, edata_hbm, out_hbm, acc_sh, fstage_sh, *bufs):
    ebuf = bufs[0:NEB]
    rows = bufs[NEB:NEB + NRB]
    msg = bufs[NEB + NRB:NEB + NRB + NMB]
    sems = bufs[NEB + NRB + NMB:]
    esem = sems[0:NEB]
    gsem = sems[NEB:NEB + NRB]
    ssem = sems[NEB + NRB:NEB + NRB + NMB]
    zsem = sems[NEB + NRB + NMB]

    c = lax.axis_index("c")
    s = lax.axis_index("s")
    ed_s = edata_hbm.at[s]
    fT_c = fT_hbm.at[c]
    last = NCHUNK - 1

    def load_edata(b, slot):
        pltpu.async_copy(ed_s.at[b], ebuf[slot], esem[slot])

    def wait_edata(slot):
        pltpu.make_async_copy(ed_s.at[0], ebuf[slot], esem[slot]).wait()

    def gather(b_slot, r):
        pltpu.async_copy(fstage_sh.at[ebuf[b_slot].at[0]], rows[r], gsem[r])

    def wait_gather(r):
        pltpu.make_async_copy(fstage_sh.at[ebuf[0].at[0]], rows[r], gsem[r]
                              ).wait()

    def scatter(slot_e, p):
        pltpu.async_copy(msg[p], acc_sh.at[ebuf[slot_e].at[1]], ssem[p],
                         add=True)

    def wait_scatter(p):
        pltpu.make_async_copy(msg[p], acc_sh.at[ebuf[0].at[1]], ssem[p]
                              ).wait()

    def compute(slot_e, r, p):
        rw = rows[r]
        mg = msg[p]

        @plsc.parallel_loop(0, K // 16, unroll=5)
        def _groups(j):
            w16 = plsc.bitcast(ebuf[slot_e][2, pl.ds(j * 16, 16)],
                               jnp.float32)
            for kk in range(16):
                k = j * 16 + kk
                w = w16[kk]
                for blk in range(DH // 32):
                    vv = rw[k, pl.ds(blk * 32, 32)]
                    a, b = plsc.unpack(
                        vv, format=plsc.PackFormat.INTERLEAVED,
                        preferred_element_type=jnp.float32)
                    m1a = a * w
                    m1b = b * w
                    mg[k, pl.ds(blk * 32, 32)] = plsc.pack(
                        m1a, m1b, format=plsc.PackFormat.INTERLEAVED)
                    mg[k, pl.ds(DH + blk * 32, 32)] = plsc.pack(
                        m1a * a, m1b * b, format=plsc.PackFormat.INTERLEAVED)

    for slot in range(4):
        load_edata(slot, slot)

    row0 = s * ROWS_PER_TILE
    fcp = pltpu.async_copy(fT_c.at[pl.ds(row0, ROWS_PER_TILE)],
                           fstage_sh.at[pl.ds(row0, ROWS_PER_TILE)], zsem)

    def zrow(r_, _):
        for dd in range(D // 32):
            msg[0][r_, pl.ds(dd * 32, 32)] = jnp.zeros((32,), jnp.bfloat16)
        return _
    lax.fori_loop(0, K, zrow, 0)
    fcp.wait()
    zcps = [
        pltpu.async_copy(msg[0], acc_sh.at[pl.ds(row0 + j * K, K)], zsem)
        for j in range(ROWS_PER_TILE // K)
    ]
    for z in zcps:
        z.wait()
    plsc.subcore_barrier()

    for u in range(2):
        wait_edata(u)
        gather(u, u)

    def six_body(i6, _):
        b0 = i6 * 6
        for u in range(6):
            b = b0 + u
            r = u % 2
            p = u % 2
            wait_gather(r)
            if u >= 2:
                wait_scatter(p)
            else:
                @pl.when(i6 > 0)
                def _ws():
                    wait_scatter(p)
            load_edata(jnp.minimum(b + 4, last), (u + 4) % 6)
            compute(u, r, p)
            scatter(u, p)
            wait_edata((u + 2) % 6)
            gather((u + 2) % 6, r)
        return _

    lax.fori_loop(0, NCHUNK // 6, six_body, 0)

    for r in range(2):
        wait_gather(r)
    wait_scatter(0)
    wait_scatter(1)
    wait_edata(2)
    wait_edata(3)

    plsc.subcore_barrier()
    pltpu.sync_copy(acc_sh.at[pl.ds(row0, ROWS_PER_TILE)],
                    out_hbm.at[c].at[pl.ds(row0, ROWS_PER_TILE)])


@jax.jit
def _spmm_sc(fT, edata):
    mesh = plsc.VectorSubcoreMesh(core_axis_name="c", subcore_axis_name="s")
    run = pl.kernel(
        _sc_body,
        out_type=jax.ShapeDtypeStruct((NUM_CORES, NP, D), jnp.bfloat16),
        mesh=mesh,
        scratch_types=(
            [pltpu.VMEM_SHARED((NP, D), jnp.bfloat16)]
            + [pltpu.VMEM_SHARED((NP, DH), jnp.bfloat16)]
            + [pltpu.VMEM((3, K), jnp.int32)] * NEB
            + [pltpu.VMEM((K, DH), jnp.bfloat16)] * NRB
            + [pltpu.VMEM((K, D), jnp.bfloat16)] * NMB
            + [pltpu.SemaphoreType.DMA] * (NEB + NRB + NMB + 1)
        ),
        compiler_params=pltpu.CompilerParams(use_tc_tiling_on_sc=False,
                                             needs_layout_passes=False),
    )
    return run(fT, edata)


def _tc_body(f_ref, acc_ref, w1_ref, wsc_ref, b1_ref, b2_ref, o_ref):
    y = jnp.dot(f_ref[...], w1_ref[...], preferred_element_type=jnp.float32)
    y += jnp.dot(acc_ref[0], wsc_ref[0], preferred_element_type=jnp.float32)
    y += jnp.dot(acc_ref[1], wsc_ref[1], preferred_element_type=jnp.float32)
    o_ref[...] = y + b1_ref[...] + b2_ref[...]


@jax.jit
def _epilogue_tc(features, acc, W1, Wsc, b1, b2):
    R = 1000
    grid = (N // R,)
    return pl.pallas_call(
        _tc_body,
        grid=grid,
        in_specs=[
            pl.BlockSpec((R, D), lambda i: (i, 0)),
            pl.BlockSpec((NUM_CORES, R, D), lambda i: (0, i, 0)),
            pl.BlockSpec((D, D), lambda i: (0, 0)),
            pl.BlockSpec((NUM_CORES, D, D), lambda i: (0, 0, 0)),
            pl.BlockSpec((1, D), lambda i: (0, 0)),
            pl.BlockSpec((1, D), lambda i: (0, 0)),
        ],
        out_specs=pl.BlockSpec((R, D), lambda i: (i, 0)),
        out_shape=jax.ShapeDtypeStruct((N, D), jnp.float32),
    )(features, acc, W1, Wsc, b1, b2)


def kernel(features, edge_index, edge_weight, W1, b1, W2, b2):
    fT = (features.astype(jnp.bfloat16)
          .reshape(N, NUM_CORES, DH).transpose(1, 0, 2))
    fT = jnp.pad(fT, ((0, 0), (0, NP - N), (0, 0)))
    wbits = lax.bitcast_convert_type(edge_weight, jnp.int32)
    packed = jnp.stack([edge_index[0], edge_index[1], wbits], axis=0)
    packed = jnp.pad(packed, ((0, 0), (0, E_PAD - E)))
    edata = (packed.reshape(3, NUM_SUBCORES, NCHUNK, K)
             .transpose(1, 2, 0, 3))
    acc = _spmm_sc(fT, edata)
    Wsc = jnp.stack([
        jnp.concatenate([W1[:DH], W2[:DH]], axis=0),
        jnp.concatenate([W1[DH:], W2[DH:]], axis=0),
    ]).astype(jnp.bfloat16)
    return _epilogue_tc(features, acc, W1, Wsc,
                        b1.reshape(1, D), b2.reshape(1, D))


# speedup vs baseline: 1.9491x; 1.0208x over previous
import jax
import jax.numpy as jnp
from jax import lax
from jax.experimental import pallas as pl
from jax.experimental.pallas import tpu as pltpu
from jax.experimental.pallas import tpu_sc as plsc

N = 10000
E = 320000
D = 128
DH = D // 2

NUM_CORES = 2
NUM_SUBCORES = 16
K = 128
NCHUNK = 162
E_PAD = NUM_SUBCORES * NCHUNK * K
NP = 10240
ROWS_PER_TILE = NP // NUM_SUBCORES

NEB = 6
NRB = 2
NMB = 2


def _sc_body(fT_hbm, edata_hbm, out_hbm, acc_sh, fstage_sh, *bufs):
    ebuf = bufs[0:NEB]
    rows = bufs[NEB:NEB + NRB]
    msg = bufs[NEB + NRB:NEB + NRB + NMB]
    sems = bufs[NEB + NRB + NMB:]
    esem = sems[0:NEB]
    gsem = sems[NEB:NEB + NRB]
    ssem = sems[NEB + NRB:NEB + NRB + NMB]
    zsem = sems[NEB + NRB + NMB]

    c = lax.axis_index("c")
    s = lax.axis_index("s")
    ed_s = edata_hbm.at[s]
    fT_c = fT_hbm.at[c]
    last = NCHUNK - 1

    def load_edata(b, slot):
        pltpu.async_copy(ed_s.at[b], ebuf[slot], esem[slot])

    def wait_edata(slot):
        pltpu.make_async_copy(ed_s.at[0], ebuf[slot], esem[slot]).wait()

    def gather(b_slot, r):
        pltpu.async_copy(fstage_sh.at[ebuf[b_slot].at[0]], rows[r], gsem[r])

    def wait_gather(r):
        pltpu.make_async_copy(fstage_sh.at[ebuf[0].at[0]], rows[r], gsem[r]
                              ).wait()

    def scatter(slot_e, p):
        pltpu.async_copy(msg[p], acc_sh.at[ebuf[slot_e].at[1]], ssem[p],
                         add=True)

    def wait_scatter(p):
        pltpu.make_async_copy(msg[p], acc_sh.at[ebuf[0].at[1]], ssem[p]
                              ).wait()

    def compute(slot_e, r, p):
        rw = rows[r]
        mg = msg[p]

        @plsc.parallel_loop(0, K // 16, unroll=8)
        def _groups(j):
            w16 = plsc.bitcast(ebuf[slot_e][2, pl.ds(j * 16, 16)],
                               jnp.float32)
            for kk in range(16):
                k = j * 16 + kk
                w = w16[kk]
                for blk in range(DH // 32):
                    vv = rw[k, pl.ds(blk * 32, 32)]
                    a, b = plsc.unpack(
                        vv, format=plsc.PackFormat.INTERLEAVED,
                        preferred_element_type=jnp.float32)
                    m1a = a * w
                    m1b = b * w
                    mg[k, pl.ds(blk * 32, 32)] = plsc.pack(
                        m1a, m1b, format=plsc.PackFormat.INTERLEAVED)
                    mg[k, pl.ds(DH + blk * 32, 32)] = plsc.pack(
                        m1a * a, m1b * b, format=plsc.PackFormat.INTERLEAVED)

    for slot in range(4):
        load_edata(slot, slot)

    row0 = s * ROWS_PER_TILE
    fcp = pltpu.async_copy(fT_c.at[pl.ds(row0, ROWS_PER_TILE)],
                           fstage_sh.at[pl.ds(row0, ROWS_PER_TILE)], zsem)

    def zrow(r_, _):
        for dd in range(D // 32):
            msg[0][r_, pl.ds(dd * 32, 32)] = jnp.zeros((32,), jnp.bfloat16)
        return _
    lax.fori_loop(0, K, zrow, 0)
    fcp.wait()
    zcps = [
        pltpu.async_copy(msg[0], acc_sh.at[pl.ds(row0 + j * K, K)], zsem)
        for j in range(ROWS_PER_TILE // K)
    ]
    for z in zcps:
        z.wait()
    plsc.subcore_barrier()

    for u in range(2):
        wait_edata(u)
        gather(u, u)

    def six_body(i6, _):
        b0 = i6 * 6
        for u in range(6):
            b = b0 + u
            r = u % 2
            p = u % 2
            wait_gather(r)
            if u >= 2:
                wait_scatter(p)
            else:
                @pl.when(i6 > 0)
                def _ws():
                    wait_scatter(p)
            load_edata(jnp.minimum(b + 4, last), (u + 4) % 6)
            compute(u, r, p)
            scatter(u, p)
            wait_edata((u + 2) % 6)
            gather((u + 2) % 6, r)
        return _

    lax.fori_loop(0, NCHUNK // 6, six_body, 0)

    for r in range(2):
        wait_gather(r)
    wait_scatter(0)
    wait_scatter(1)
    wait_edata(2)
    wait_edata(3)

    plsc.subcore_barrier()
    pltpu.sync_copy(acc_sh.at[pl.ds(row0, ROWS_PER_TILE)],
                    out_hbm.at[c].at[pl.ds(row0, ROWS_PER_TILE)])


@jax.jit
def _spmm_sc(fT, edata):
    mesh = plsc.VectorSubcoreMesh(core_axis_name="c", subcore_axis_name="s")
    run = pl.kernel(
        _sc_body,
        out_type=jax.ShapeDtypeStruct((NUM_CORES, NP, D), jnp.bfloat16),
        mesh=mesh,
        scratch_types=(
            [pltpu.VMEM_SHARED((NP, D), jnp.bfloat16)]
            + [pltpu.VMEM_SHARED((NP, DH), jnp.bfloat16)]
            + [pltpu.VMEM((3, K), jnp.int32)] * NEB
            + [pltpu.VMEM((K, DH), jnp.bfloat16)] * NRB
            + [pltpu.VMEM((K, D), jnp.bfloat16)] * NMB
            + [pltpu.SemaphoreType.DMA] * (NEB + NRB + NMB + 1)
        ),
        compiler_params=pltpu.CompilerParams(use_tc_tiling_on_sc=False,
                                             needs_layout_passes=False),
    )
    return run(fT, edata)


def _tc_body(f_ref, acc_ref, w1_ref, wsc_ref, b1_ref, b2_ref, o_ref):
    y = jnp.dot(f_ref[...], w1_ref[...], preferred_element_type=jnp.float32)
    y += jnp.dot(acc_ref[0], wsc_ref[0], preferred_element_type=jnp.float32)
    y += jnp.dot(acc_ref[1], wsc_ref[1], preferred_element_type=jnp.float32)
    o_ref[...] = y + b1_ref[...] + b2_ref[...]


@jax.jit
def _epilogue_tc(features, acc, W1, Wsc, b1, b2):
    R = 1000
    grid = (N // R,)
    return pl.pallas_call(
        _tc_body,
        grid=grid,
        in_specs=[
            pl.BlockSpec((R, D), lambda i: (i, 0)),
            pl.BlockSpec((NUM_CORES, R, D), lambda i: (0, i, 0)),
            pl.BlockSpec((D, D), lambda i: (0, 0)),
            pl.BlockSpec((NUM_CORES, D, D), lambda i: (0, 0, 0)),
            pl.BlockSpec((1, D), lambda i: (0, 0)),
            pl.BlockSpec((1, D), lambda i: (0, 0)),
        ],
        out_specs=pl.BlockSpec((R, D), lambda i: (i, 0)),
        out_shape=jax.ShapeDtypeStruct((N, D), jnp.float32),
    )(features, acc, W1, Wsc, b1, b2)


def kernel(features, edge_index, edge_weight, W1, b1, W2, b2):
    fT = (features.astype(jnp.bfloat16)
          .reshape(N, NUM_CORES, DH).transpose(1, 0, 2))
    fT = jnp.pad(fT, ((0, 0), (0, NP - N), (0, 0)))
    wbits = lax.bitcast_convert_type(edge_weight, jnp.int32)
    packed = jnp.stack([edge_index[0], edge_index[1], wbits], axis=0)
    packed = jnp.pad(packed, ((0, 0), (0, E_PAD - E)))
    edata = (packed.reshape(3, NUM_SUBCORES, NCHUNK, K)
             .transpose(1, 2, 0, 3))
    acc = _spmm_sc(fT, edata)
    Wsc = jnp.stack([
        jnp.concatenate([W1[:DH], W2[:DH]], axis=0),
        jnp.concatenate([W1[DH:], W2[DH:]], axis=0),
    ]).astype(jnp.bfloat16)
    return _epilogue_tc(features, acc, W1, Wsc,
                        b1.reshape(1, D), b2.reshape(1, D))
